# baseline (speedup 1.0000x reference)
import functools

import jax
import jax.numpy as jnp
from jax import lax
from jax.experimental import pallas as pl
from jax.experimental.pallas import tpu as pltpu
from jax.experimental.pallas import tpu_sc as plsc

VOCAB = 1000000
HIDDEN = 64
SEQ_LEN = 200
BATCH = 4096
N = BATCH * SEQ_LEN
NC, NS, LANES = 2, 16, 16
NW = NC * NS
PER_W = N // NW
CHUNK = 512
NCHUNK = PER_W // CHUNK
NGROUP = CHUNK // LANES
GATHER_SLICE = 128
LN_EPS = 1e-8


def _rsqrt(x):
    i = lax.bitcast_convert_type(x, jnp.int32)
    i = jnp.int32(0x5F3759DF) - lax.shift_right_logical(i, 1)
    y = lax.bitcast_convert_type(i, jnp.float32)
    for _ in range(3):
        y = y * (1.5 - 0.5 * x * y * y)
    return y


def _emb_body(tok_hbm, word_hbm, pos_hbm, gam_hbm, bet_hbm, out_hbm,
              idx_v, rows_v, pos_v, xbuf, gb, bb, gv, bv, sem):
    wid = lax.axis_index("s") * NC + lax.axis_index("c")
    base = wid * PER_W

    pltpu.sync_copy(pos_hbm, pos_v)
    pltpu.sync_copy(gam_hbm, gv)
    pltpu.sync_copy(bet_hbm, bv)

    for h in range(HIDDEN):
        hv = jnp.full((LANES,), h, jnp.int32)
        gb[pl.ds(h * LANES, LANES)] = plsc.load_gather(gv, [hv])
        bb[pl.ds(h * LANES, LANES)] = plsc.load_gather(bv, [hv])

    def chunk_body(c, carry):
        fbase = base + c * CHUNK
        pltpu.sync_copy(tok_hbm.at[pl.ds(fbase, CHUNK)], idx_v)
        copies = []
        for j in range(CHUNK // GATHER_SLICE):
            copies.append(pltpu.async_copy(
                word_hbm.at[idx_v.at[pl.ds(j * GATHER_SLICE, GATHER_SLICE)]],
                rows_v.at[pl.ds(j * GATHER_SLICE, GATHER_SLICE)],
                sem))
        for cp in copies:
            cp.wait()

        def group_body(g, carry2):
            lane = lax.iota(jnp.int32, LANES)
            tok = idx_v[pl.ds(g * LANES, LANES)]
            rows16 = g * LANES + lane
            l_idx = jnp.remainder(fbase + g * LANES + lane,
                                  jnp.int32(SEQ_LEN))
            s = jnp.zeros((LANES,), jnp.float32)
            ss = jnp.zeros((LANES,), jnp.float32)
            for h in range(HIDDEN):
                hv = jnp.full((LANES,), h, jnp.int32)
                w = plsc.load_gather(rows_v, [rows16, hv])
                p = plsc.load_gather(pos_v, [l_idx, hv])
                x = w + p
                xbuf[pl.ds(h * LANES, LANES)] = x
                s = s + x
                ss = ss + x * x
            mean = s * (1.0 / HIDDEN)
            var = ss * (1.0 / HIDDEN) - mean * mean
            rs = _rsqrt(var + LN_EPS)
            msk = tok != 0
            for h in range(HIDDEN):
                x = xbuf[pl.ds(h * LANES, LANES)]
                y = (x - mean) * rs * gb[pl.ds(h * LANES, LANES)] \
                    + bb[pl.ds(h * LANES, LANES)]
                y = jnp.where(msk, y, 0.0)
                plsc.store_scatter(rows_v,
                                   [rows16, jnp.full((LANES,), h, jnp.int32)],
                                   y)
            return carry2

        lax.fori_loop(0, NGROUP, group_body, 0)
        pltpu.sync_copy(rows_v, out_hbm.at[pl.ds(fbase, CHUNK)])
        return carry

    lax.fori_loop(0, NCHUNK, chunk_body, 0)


_emb = functools.partial(
    pl.kernel,
    out_type=jax.ShapeDtypeStruct((N, HIDDEN), jnp.float32),
    mesh=plsc.VectorSubcoreMesh(core_axis_name="c", subcore_axis_name="s",
                                num_cores=NC, num_subcores=NS),
    compiler_params=pltpu.CompilerParams(needs_layout_passes=False,
                                         use_tc_tiling_on_sc=False),
    scratch_types=[
        pltpu.VMEM((CHUNK,), jnp.int32),
        pltpu.VMEM((CHUNK, HIDDEN), jnp.float32),
        pltpu.VMEM((SEQ_LEN, HIDDEN), jnp.float32),
        pltpu.VMEM((HIDDEN * LANES,), jnp.float32),
        pltpu.VMEM((HIDDEN * LANES,), jnp.float32),
        pltpu.VMEM((HIDDEN * LANES,), jnp.float32),
        pltpu.VMEM((HIDDEN,), jnp.float32),
        pltpu.VMEM((HIDDEN,), jnp.float32),
        pltpu.SemaphoreType.DMA,
    ],
)(_emb_body)


@jax.jit
def kernel(tokens, word_table, pos_table, gamma, beta):
    tok_flat = tokens.reshape(-1).astype(jnp.int32)
    out = _emb(tok_flat, word_table, pos_table, gamma, beta)
    return out.reshape(BATCH, SEQ_LEN, HIDDEN)


# speedup vs baseline: 3.3041x; 3.3041x over previous
import functools

import jax
import jax.numpy as jnp
from jax import lax
from jax.experimental import pallas as pl
from jax.experimental.pallas import tpu as pltpu
from jax.experimental.pallas import tpu_sc as plsc

VOCAB = 1000000
HIDDEN = 64
SEQ_LEN = 200
BATCH = 4096
N = BATCH * SEQ_LEN
NC, NS, LANES = 2, 16, 16
NW = NC * NS
PER_W = N // NW
CHUNK = 512
NCHUNK = PER_W // CHUNK
HQ = HIDDEN // LANES
GATHER_SLICE = 128
LN_EPS = 1e-8


def _rsqrt(x):
    i = lax.bitcast_convert_type(x, jnp.int32)
    i = jnp.int32(0x5F3759DF) - lax.shift_right_logical(i, 1)
    y = lax.bitcast_convert_type(i, jnp.float32)
    for _ in range(3):
        y = y * (1.5 - 0.5 * x * y * y)
    return y


def _emb_body(tok_hbm, word_hbm, pos_hbm, gam_hbm, bet_hbm, out_hbm,
              idx_v, rows_v, out_v, pos_v, gv, bv, sem):
    wid = lax.axis_index("s") * NC + lax.axis_index("c")
    base = wid * PER_W

    pltpu.sync_copy(pos_hbm, pos_v)
    pltpu.sync_copy(gam_hbm, gv)
    pltpu.sync_copy(bet_hbm, bv)

    gvec = [gv[pl.ds(i * LANES, LANES)] for i in range(HQ)]
    bvec = [bv[pl.ds(i * LANES, LANES)] for i in range(HQ)]

    def chunk_body(c, carry):
        fbase = base + c * CHUNK
        pltpu.sync_copy(tok_hbm.at[pl.ds(fbase, CHUNK)],
                        idx_v.at[pl.ds(0, CHUNK)])
        copies = []
        for j in range(CHUNK // GATHER_SLICE):
            copies.append(pltpu.async_copy(
                word_hbm.at[idx_v.at[pl.ds(j * GATHER_SLICE, GATHER_SLICE)]],
                rows_v.at[pl.ds(j * GATHER_SLICE, GATHER_SLICE)],
                sem))
        for cp in copies:
            cp.wait()

        @plsc.parallel_loop(0, CHUNK, 1, unroll=4)
        def token_body(t):
            l = jnp.remainder(fbase + t, jnp.int32(SEQ_LEN))
            x = [rows_v[t, pl.ds(i * LANES, LANES)]
                 + pos_v[pl.ds(l * HIDDEN + i * LANES, LANES)]
                 for i in range(HQ)]
            s = (x[0] + x[1]) + (x[2] + x[3])
            ss = (x[0] * x[0] + x[1] * x[1]) + (x[2] * x[2] + x[3] * x[3])
            tot = jnp.sum(s)
            tot2 = jnp.sum(ss)
            mean = tot * (1.0 / HIDDEN)
            var = tot2 * (1.0 / HIDDEN) - mean * mean
            rs = _rsqrt(var + LN_EPS)
            tok = idx_v[pl.ds(t, LANES)][0]
            msk = jnp.where(tok != 0, jnp.float32(1.0), jnp.float32(0.0))
            rsm = rs * msk
            for i in range(HQ):
                y = (x[i] - mean) * rsm * gvec[i] + msk * bvec[i]
                out_v[t, pl.ds(i * LANES, LANES)] = y

        pltpu.sync_copy(out_v, out_hbm.at[pl.ds(fbase, CHUNK)])
        return carry

    lax.fori_loop(0, NCHUNK, chunk_body, 0)


_emb = functools.partial(
    pl.kernel,
    out_type=jax.ShapeDtypeStruct((N, HIDDEN), jnp.float32),
    mesh=plsc.VectorSubcoreMesh(core_axis_name="c", subcore_axis_name="s",
                                num_cores=NC, num_subcores=NS),
    compiler_params=pltpu.CompilerParams(needs_layout_passes=False,
                                         use_tc_tiling_on_sc=False),
    scratch_types=[
        pltpu.VMEM((CHUNK + LANES,), jnp.int32),
        pltpu.VMEM((CHUNK, HIDDEN), jnp.float32),
        pltpu.VMEM((CHUNK, HIDDEN), jnp.float32),
        pltpu.VMEM((SEQ_LEN * HIDDEN,), jnp.float32),
        pltpu.VMEM((HIDDEN,), jnp.float32),
        pltpu.VMEM((HIDDEN,), jnp.float32),
        pltpu.SemaphoreType.DMA,
    ],
)(_emb_body)


@jax.jit
def kernel(tokens, word_table, pos_table, gamma, beta):
    tok_flat = tokens.reshape(-1).astype(jnp.int32)
    out = _emb(tok_flat, word_table, pos_table.reshape(-1), gamma, beta)
    return out.reshape(BATCH, SEQ_LEN, HIDDEN)
